# baseline (speedup 1.0000x reference)
import functools

import jax
import jax.numpy as jnp
import numpy as np
from jax import lax
from jax.experimental import pallas as pl
from jax.experimental.pallas import tpu as pltpu
from jax.experimental.pallas import tpu_sc as plsc

_EOS = 3
_NCMD = 6
_NARGS = 16
_ADIM = 257
_EW = [float(np.exp(-2.0 * abs(k))) for k in range(-3, 4)]

_B, _S = 128, 64
_BT = 64
_NW = 32
_SC_POS0 = _BT * _S * _NARGS
_SC_NPOS = (_B - _BT) * _S * _NARGS
_PER_W = _SC_NPOS // _NW
_CHUNK = 128
_NCH = _PER_W // _CHUNK


def _tc_body(t_ref, cmdm_ref, cmdl_ref, cl_ref, x_ref, part_ref, acc_ref):
    i = pl.program_id(0)
    n = pl.num_programs(0)
    x = x_ref[...]
    t = t_ref[...] + 1

    e = jnp.exp(x)
    s = jnp.sum(e, axis=-1)

    c = jax.lax.broadcasted_iota(jnp.int32, x.shape, 3)
    ad = jnp.abs(c - t[..., None])
    w = jnp.where(ad <= 3, jnp.exp(-2.0 * ad.astype(jnp.float32)), 0.0)
    z = jnp.sum(w, axis=-1)
    g = jnp.sum(w * x, axis=-1)

    tf = t.astype(jnp.float32)
    delta = jnp.where(t >= 253, _EW[0] - jnp.exp(-2.0 * (256.0 - tf)), 0.0)
    z = z + delta
    g = g + delta * x[..., 256]

    cmdb = cmdm_ref[...][0][..., None]
    a = jax.lax.broadcasted_iota(jnp.int32, t.shape, 2)
    mask = (((cmdb == 0) & (a < 2)) |
            ((cmdb == 1) & (a < 4)) |
            ((cmdb == 2) & ((a < 2) | (a == 4))) |
            ((cmdb == 5) & (a >= 5))).astype(jnp.float32)

    la = jnp.sum(mask * (jnp.log(s) - g / z))
    da = jnp.sum(mask)

    cl = cl_ref[...]
    cmdf = cmdl_ref[...][0]
    eos = (cmdf == _EOS).astype(jnp.float32)
    sdim = cmdf.shape[1]
    r = jax.lax.broadcasted_iota(jnp.int32, (sdim, sdim), 0)
    cc = jax.lax.broadcasted_iota(jnp.int32, (sdim, sdim), 1)
    tri = (r < cc).astype(jnp.float32)
    excl = jnp.dot(eos, tri, preferred_element_type=jnp.float32)
    pad0 = (excl == 0.0).astype(jnp.float32)
    vis = (jnp.sum(eos, axis=1) < float(sdim)).astype(jnp.float32)
    pad = pad0 * vis[:, None]
    mx = jnp.max(cl, axis=-1)
    lse6 = mx + jnp.log(jnp.sum(jnp.exp(cl - mx[..., None]), axis=-1))
    c6 = jax.lax.broadcasted_iota(jnp.int32, cl.shape, 2)
    picked = jnp.sum(jnp.where(c6 == cmdf[..., None], cl, 0.0), axis=-1)
    nll = lse6 - picked
    lc = jnp.sum(pad * nll)
    dc = jnp.sum(pad)

    @pl.when(i == 0)
    def _():
        acc_ref[0] = la
        acc_ref[1] = da
        acc_ref[2] = lc
        acc_ref[3] = dc

    @pl.when(i != 0)
    def _():
        acc_ref[0] += la
        acc_ref[1] += da
        acc_ref[2] += lc
        acc_ref[3] += dc

    @pl.when(i == n - 1)
    def _():
        part_ref[0, 0] = acc_ref[0]
        part_ref[0, 1] = acc_ref[1]
        part_ref[0, 2] = acc_ref[2]
        part_ref[0, 3] = acc_ref[3]


def _sc_body(xflat_ref, aflat_ref, cflat_ref, sm_ref, pp_ref,
             xbuf, abuf, cbuf, sbuf, pbuf):
    wid = lax.axis_index("s") * 2 + lax.axis_index("c")
    base = pl.multiple_of(_SC_POS0 + wid * _PER_W, _CHUNK)
    lane = lax.iota(jnp.int32, 16)
    lanef = lane.astype(jnp.float32)
    zero = jnp.zeros((16,), jnp.float32)

    def _chunk(ch, carry):
        cb = pl.multiple_of(base + ch * _CHUNK, _CHUNK)
        pltpu.sync_copy(
            xflat_ref.at[pl.ds(pl.multiple_of(cb * _ADIM, 8),
                               _CHUNK * _ADIM)], xbuf)
        pltpu.sync_copy(aflat_ref.at[pl.ds(cb, _CHUNK)], abuf)
        pltpu.sync_copy(
            cflat_ref.at[pl.ds(pl.multiple_of(cb // _NARGS, 8),
                               _CHUNK // _NARGS)],
            cbuf.at[pl.ds(0, _CHUNK // _NARGS)])

        def _group(j, carry2):
            p1a, daa = carry2
            jb = pl.multiple_of(16 * j, 16)
            t = abuf[pl.ds(jb, 16)] + 1
            cmd = plsc.load_gather(cbuf, [lane * 0 + j])
            lanebase = (16 * j + lane) * _ADIM

            maskb = (((cmd == 0) & (lane < 2)) |
                     ((cmd == 1) & (lane < 4)) |
                     ((cmd == 2) & ((lane < 2) | (lane == 4))) |
                     ((cmd == 5) & (lane >= 5)))
            maskf = jnp.where(maskb, 1.0, 0.0)

            def _cls(it, accs):
                a0, a1, a2, a3 = accs
                vals = []
                for u in range(8):
                    v = plsc.load_gather(xbuf, [lanebase + (8 * it + u)])
                    vals.append(jnp.exp(v))
                a0 = a0 + vals[0] + vals[4]
                a1 = a1 + vals[1] + vals[5]
                a2 = a2 + vals[2] + vals[6]
                a3 = a3 + vals[3] + vals[7]
                return (a0, a1, a2, a3)

            accs = lax.fori_loop(0, 32, _cls, (zero, zero, zero, zero))
            v256 = plsc.load_gather(xbuf, [lanebase + 256])
            s = accs[0] + accs[1] + accs[2] + accs[3] + jnp.exp(v256)

            g = zero
            z = zero
            for k in range(7):
                tpk = t + (k - 3)
                st = jnp.clip(tpk, 0, _ADIM - 1)
                if k == 6:
                    surv = tpk >= 0
                else:
                    surv = (tpk >= 0) & (tpk <= 255)
                wk = jnp.where(surv, _EW[k], 0.0)
                vk = plsc.load_gather(xbuf, [lanebase + st])
                g = g + wk * vk
                z = z + wk
            sbuf[pl.ds(jb, 16)] = jnp.where(maskb, s, 1.0)
            return (p1a + maskf * (g / z), daa + maskf)

        carry = lax.fori_loop(0, _CHUNK // 16, _group, carry)
        pltpu.sync_copy(
            sbuf, sm_ref.at[pl.ds(pl.multiple_of(cb - _SC_POS0, 8),
                                  _CHUNK)])
        return carry

    p1acc, daacc = lax.fori_loop(0, _NCH, _chunk, (zero, zero))

    pbuf[pl.ds(0, 16)] = p1acc
    pbuf[pl.ds(16, 16)] = daacc
    pltpu.sync_copy(pbuf,
                    pp_ref.at[pl.ds(pl.multiple_of(wid * 32, 8), 32)])


def _combine_body(part_ref, pp_ref, sm_ref, oc_ref, oa_ref):
    logs = jnp.log(sm_ref[...])
    pp = pp_ref[...]
    r = jax.lax.broadcasted_iota(jnp.int32, pp.shape, 1)
    flat = jax.lax.broadcasted_iota(jnp.int32, pp.shape, 0) * 128 + r
    is_p1 = ((flat // 16) % 2) == 0
    p1_sc = jnp.sum(jnp.where(is_p1, pp, 0.0))
    da_sc = jnp.sum(jnp.where(is_p1, 0.0, pp))
    la = part_ref[0, 0] + jnp.sum(logs) - p1_sc
    da = part_ref[0, 1] + da_sc
    oc_ref[0, 0] = part_ref[0, 2] / part_ref[0, 3]
    oa_ref[0, 0] = 2.0 * la / da


@jax.jit
def kernel(command_logits, args_logits, command, args):
    bsz, sdim = command.shape
    bb = 4
    grid = _BT // bb
    bc = bsz // grid

    xflat = args_logits.reshape(-1)
    aflat = args.reshape(-1)
    cflat = command.reshape(-1)

    part = pl.pallas_call(
        _tc_body,
        grid=(grid,),
        in_specs=[
            pl.BlockSpec((bb, sdim, _NARGS), lambda i: (i, 0, 0)),
            pl.BlockSpec((1, bb, sdim), lambda i: (i, 0, 0)),
            pl.BlockSpec((1, bc, sdim), lambda i: (i, 0, 0)),
            pl.BlockSpec((bc, sdim, _NCMD), lambda i: (i, 0, 0)),
            pl.BlockSpec((bb, sdim, _NARGS, _ADIM), lambda i: (i, 0, 0, 0)),
        ],
        out_specs=pl.BlockSpec((1, 4), lambda i: (0, 0),
                               memory_space=pltpu.SMEM),
        out_shape=jax.ShapeDtypeStruct((1, 4), jnp.float32),
        scratch_shapes=[pltpu.SMEM((4,), jnp.float32)],
        compiler_params=pltpu.CompilerParams(
            dimension_semantics=("arbitrary",)),
    )(args, command.reshape(bsz // bb, bb, sdim),
      command.reshape(grid, bc, sdim), command_logits, args_logits)

    mesh = plsc.VectorSubcoreMesh(core_axis_name="c", subcore_axis_name="s")
    sc_call = functools.partial(
        pl.kernel, mesh=mesh,
        out_type=[
            jax.ShapeDtypeStruct((_SC_NPOS,), jnp.float32),
            jax.ShapeDtypeStruct((_NW * 32,), jnp.float32),
        ],
        scratch_types=[
            pltpu.VMEM((_CHUNK * _ADIM,), jnp.float32),
            pltpu.VMEM((_CHUNK,), jnp.int32),
            pltpu.VMEM((16,), jnp.int32),
            pltpu.VMEM((_CHUNK,), jnp.float32),
            pltpu.VMEM((32,), jnp.float32),
        ],
        compiler_params=pltpu.CompilerParams(needs_layout_passes=False),
    )(_sc_body)
    sm, pp = sc_call(xflat, aflat, cflat)

    scalar_spec0 = pl.BlockSpec((1, 1), lambda: (0, 0),
                                memory_space=pltpu.SMEM)
    oc, oa = pl.pallas_call(
        _combine_body,
        in_specs=[
            pl.BlockSpec((1, 4), lambda: (0, 0), memory_space=pltpu.SMEM),
            pl.BlockSpec((_NW * 32 // 128, 128), lambda: (0, 0)),
            pl.BlockSpec((_SC_NPOS // 128, 128), lambda: (0, 0)),
        ],
        out_specs=[scalar_spec0] * 2,
        out_shape=[jax.ShapeDtypeStruct((1, 1), jnp.float32)] * 2,
    )(part, pp.reshape(_NW * 32 // 128, 128),
      sm.reshape(_SC_NPOS // 128, 128))

    return (oc[0, 0], oa[0, 0])


# speedup vs baseline: 1.2418x; 1.2418x over previous
import functools

import jax
import jax.numpy as jnp
import numpy as np
from jax.experimental import pallas as pl
from jax.experimental.pallas import tpu as pltpu

_EOS = 3
_NCMD = 6
_NARGS = 16
_ADIM = 257
_E2 = float(np.exp(-2.0))
_E4 = float(np.exp(-4.0))
_E6 = float(np.exp(-6.0))
_ZFULL = 1.0 + 2.0 * (_E2 + _E4 + _E6)


def _loss_body(t_ref, cmd_ref, cl_ref, x_ref, out_cmd_ref, out_args_ref,
               acc_ref):
    i = pl.program_id(0)
    n = pl.num_programs(0)
    x = x_ref[...]
    t = t_ref[...] + 1

    e = jnp.exp(x)
    s = jnp.sum(e, axis=-1)

    c = jax.lax.broadcasted_iota(jnp.int32, x.shape, 3)
    ad = jnp.abs(c - t[..., None]).astype(jnp.float32)
    w = jnp.exp(-2.0 * ad)
    g = jnp.sum(w * x, axis=-1)

    tf = t.astype(jnp.float32)
    z = (_ZFULL
         - jnp.where(t == 1, _E4 + _E6, jnp.where(t == 2, _E6, 0.0))
         - jnp.where(t == 254, _E4,
                     jnp.where(t == 255, _E2 + _E4,
                               jnp.where(t == 256, 1.0 + _E2 + _E4, 0.0))))
    g = g - jnp.where(t > 3, jnp.exp(-2.0 * tf), 0.0) * x[..., 0]
    g = g + (jnp.where(t >= 253, _E6, 0.0)
             - jnp.exp(-2.0 * (256.0 - tf))) * x[..., 256]

    cmdf = cmd_ref[...][0]
    cmdb = cmdf[..., None]
    a = jax.lax.broadcasted_iota(jnp.int32, t.shape, 2)
    mask = (((cmdb == 0) & (a < 2)) |
            ((cmdb == 1) & (a < 4)) |
            ((cmdb == 2) & ((a < 2) | (a == 4))) |
            ((cmdb == 5) & (a >= 5))).astype(jnp.float32)

    la = jnp.sum(mask * (jnp.log(s) - g / z))
    da = jnp.sum(mask)

    cl = cl_ref[...]
    eos = (cmdf == _EOS).astype(jnp.float32)
    sdim = cmdf.shape[1]
    r = jax.lax.broadcasted_iota(jnp.int32, (sdim, sdim), 0)
    cc = jax.lax.broadcasted_iota(jnp.int32, (sdim, sdim), 1)
    tri = (r < cc).astype(jnp.float32)
    excl = jnp.dot(eos, tri, preferred_element_type=jnp.float32)
    pad0 = (excl == 0.0).astype(jnp.float32)
    vis = (jnp.sum(eos, axis=1) < float(sdim)).astype(jnp.float32)
    pad = pad0 * vis[:, None]
    mx = jnp.max(cl, axis=-1)
    lse6 = mx + jnp.log(jnp.sum(jnp.exp(cl - mx[..., None]), axis=-1))
    c6 = jax.lax.broadcasted_iota(jnp.int32, cl.shape, 2)
    picked = jnp.sum(jnp.where(c6 == cmdf[..., None], cl, 0.0), axis=-1)
    nll = lse6 - picked
    lc = jnp.sum(pad * nll)
    dc = jnp.sum(pad)

    @pl.when(i == 0)
    def _():
        acc_ref[0] = la
        acc_ref[1] = da
        acc_ref[2] = lc
        acc_ref[3] = dc

    @pl.when(i != 0)
    def _():
        acc_ref[0] += la
        acc_ref[1] += da
        acc_ref[2] += lc
        acc_ref[3] += dc

    @pl.when(i == n - 1)
    def _():
        out_cmd_ref[0, 0] = acc_ref[2] / acc_ref[3]
        out_args_ref[0, 0] = 2.0 * acc_ref[0] / acc_ref[1]


@jax.jit
def kernel(command_logits, args_logits, command, args):
    bsz, sdim = command.shape
    bb = 4
    grid = bsz // bb

    scalar_spec = pl.BlockSpec((1, 1), lambda i: (0, 0),
                               memory_space=pltpu.SMEM)
    oc, oa = pl.pallas_call(
        _loss_body,
        grid=(grid,),
        in_specs=[
            pl.BlockSpec((bb, sdim, _NARGS), lambda i: (i, 0, 0)),
            pl.BlockSpec((1, bb, sdim), lambda i: (i, 0, 0)),
            pl.BlockSpec((bb, sdim, _NCMD), lambda i: (i, 0, 0)),
            pl.BlockSpec((bb, sdim, _NARGS, _ADIM), lambda i: (i, 0, 0, 0)),
        ],
        out_specs=[scalar_spec] * 2,
        out_shape=[jax.ShapeDtypeStruct((1, 1), jnp.float32)] * 2,
        scratch_shapes=[pltpu.SMEM((4,), jnp.float32)],
        compiler_params=pltpu.CompilerParams(
            dimension_semantics=("arbitrary",)),
    )(args, command.reshape(grid, bb, sdim), command_logits, args_logits)

    return (oc[0, 0], oa[0, 0])


# speedup vs baseline: 1.2840x; 1.0340x over previous
import functools

import jax
import jax.numpy as jnp
import numpy as np
from jax.experimental import pallas as pl
from jax.experimental.pallas import tpu as pltpu

_EOS = 3
_NCMD = 6
_NARGS = 16
_ADIM = 257
_EW3 = float(np.exp(-6.0))


def _loss_body(t_ref, cmd_ref, cl_ref, x_ref, out_cmd_ref, out_args_ref,
               acc_ref):
    i = pl.program_id(0)
    n = pl.num_programs(0)
    x = x_ref[...]
    t = t_ref[...] + 1

    e = jnp.exp(x)
    s = jnp.sum(e, axis=-1)

    c = jax.lax.broadcasted_iota(jnp.int32, x.shape, 3)
    ad = jnp.abs(c - t[..., None])
    w = jnp.where(ad <= 3, jnp.exp(-2.0 * ad.astype(jnp.float32)), 0.0)
    z = jnp.sum(w, axis=-1)
    g = jnp.sum(w * x, axis=-1)

    tf = t.astype(jnp.float32)
    delta = jnp.where(t >= 253, _EW3 - jnp.exp(-2.0 * (256.0 - tf)), 0.0)
    z = z + delta
    g = g + delta * x[..., 256]

    cmdf = cmd_ref[...][0]
    cmdb = cmdf[..., None]
    a = jax.lax.broadcasted_iota(jnp.int32, t.shape, 2)
    mask = (((cmdb == 0) & (a < 2)) |
            ((cmdb == 1) & (a < 4)) |
            ((cmdb == 2) & ((a < 2) | (a == 4))) |
            ((cmdb == 5) & (a >= 5))).astype(jnp.float32)

    la = jnp.sum(mask * (jnp.log(s) - g / z))
    da = jnp.sum(mask)

    cl = cl_ref[...]
    eos = (cmdf == _EOS).astype(jnp.float32)
    sdim = cmdf.shape[1]
    r = jax.lax.broadcasted_iota(jnp.int32, (sdim, sdim), 0)
    cc = jax.lax.broadcasted_iota(jnp.int32, (sdim, sdim), 1)
    tri = (r < cc).astype(jnp.float32)
    excl = jnp.dot(eos, tri, preferred_element_type=jnp.float32)
    pad0 = (excl == 0.0).astype(jnp.float32)
    vis = (jnp.sum(eos, axis=1) < float(sdim)).astype(jnp.float32)
    pad = pad0 * vis[:, None]
    mx = jnp.max(cl, axis=-1)
    lse6 = mx + jnp.log(jnp.sum(jnp.exp(cl - mx[..., None]), axis=-1))
    c6 = jax.lax.broadcasted_iota(jnp.int32, cl.shape, 2)
    picked = jnp.sum(jnp.where(c6 == cmdf[..., None], cl, 0.0), axis=-1)
    nll = lse6 - picked
    lc = jnp.sum(pad * nll)
    dc = jnp.sum(pad)

    @pl.when(i == 0)
    def _():
        acc_ref[0] = la
        acc_ref[1] = da
        acc_ref[2] = lc
        acc_ref[3] = dc

    @pl.when(i != 0)
    def _():
        acc_ref[0] += la
        acc_ref[1] += da
        acc_ref[2] += lc
        acc_ref[3] += dc

    @pl.when(i == n - 1)
    def _():
        out_cmd_ref[0, 0] = acc_ref[2] / acc_ref[3]
        out_args_ref[0, 0] = 2.0 * acc_ref[0] / acc_ref[1]


@jax.jit
def kernel(command_logits, args_logits, command, args):
    bsz, sdim = command.shape
    bb = 8
    grid = bsz // bb

    scalar_spec = pl.BlockSpec((1, 1), lambda i: (0, 0),
                               memory_space=pltpu.SMEM)
    oc, oa = pl.pallas_call(
        _loss_body,
        grid=(grid,),
        in_specs=[
            pl.BlockSpec((bb, sdim, _NARGS), lambda i: (i, 0, 0)),
            pl.BlockSpec((1, bb, sdim), lambda i: (i, 0, 0)),
            pl.BlockSpec((bb, sdim, _NCMD), lambda i: (i, 0, 0)),
            pl.BlockSpec((bb, sdim, _NARGS, _ADIM), lambda i: (i, 0, 0, 0)),
        ],
        out_specs=[scalar_spec] * 2,
        out_shape=[jax.ShapeDtypeStruct((1, 1), jnp.float32)] * 2,
        scratch_shapes=[pltpu.SMEM((4,), jnp.float32)],
        compiler_params=pltpu.CompilerParams(
            dimension_semantics=("arbitrary",)),
    )(args, command.reshape(grid, bb, sdim), command_logits, args_logits)

    return (oc[0, 0], oa[0, 0])
